# baseline (speedup 1.0000x reference)
import jax
import jax.numpy as jnp
from jax import lax
from jax.experimental import pallas as pl
from jax.experimental.pallas import tpu as pltpu
from jax.experimental.pallas import tpu_sc as plsc

_B = 4
_S = 8192
_D = 1024
_BTOT = _B * _S
_NC = 2
_NS = 16
_NW = _NC * _NS
_BPW = _BTOT // _NW
_CHUNK = 8
_NBUF = 15
_LAG = 8
_NCHUNK = _BPW // _CHUNK


def _emb_gather(pos_hbm, table_hbm, out_hbm, idx_v, rows_v, *sems):
    cid = lax.axis_index("c")
    sid = lax.axis_index("s")
    wid = sid * _NC + cid
    base = wid * _BPW
    pltpu.sync_copy(pos_hbm.at[pl.ds(base, _BPW)], idx_v)
    gsems = sems[:_NBUF]
    osems = sems[_NBUF:]

    def _aligned(off):
        return off if isinstance(off, int) else pl.multiple_of(off, _CHUNK)

    def gather_desc(g):
        b = g % _NBUF
        return pltpu.make_async_copy(
            table_hbm.at[idx_v.at[pl.ds(g * _CHUNK, _CHUNK)]],
            rows_v.at[b],
            gsems[b],
        )

    def store_desc(g):
        b = g % _NBUF
        return pltpu.make_async_copy(
            rows_v.at[b],
            out_hbm.at[pl.ds(_aligned(base + g * _CHUNK), _CHUNK)],
            osems[b],
        )

    for g in range(_NCHUNK + _LAG):
        if g < _NCHUNK:
            if g >= _NBUF:
                store_desc(g - _NBUF).wait()
            gather_desc(g).start()
        if g >= _LAG:
            gather_desc(g - _LAG).wait()
            store_desc(g - _LAG).start()
    for g in range(_NCHUNK - _NBUF, _NCHUNK):
        store_desc(g).wait()


@jax.jit
def kernel(positions, table):
    pos_flat = positions.reshape(_BTOT).astype(jnp.int32)
    mesh = plsc.VectorSubcoreMesh(core_axis_name="c", subcore_axis_name="s")
    out = pl.kernel(
        _emb_gather,
        mesh=mesh,
        out_type=jax.ShapeDtypeStruct((_BTOT, _D), jnp.float32),
        scratch_types=[
            pltpu.VMEM((_BPW,), jnp.int32),
            pltpu.VMEM((_NBUF, _CHUNK, _D), jnp.float32),
        ] + [pltpu.SemaphoreType.DMA] * (2 * _NBUF),
    )(pos_flat, table)
    return out.reshape(_B, _S, _D)


# speedup vs baseline: 1.0292x; 1.0292x over previous
import jax
import jax.numpy as jnp
from jax import lax
from jax.experimental import pallas as pl
from jax.experimental.pallas import tpu as pltpu
from jax.experimental.pallas import tpu_sc as plsc

_B = 4
_S = 8192
_D = 1024
_BTOT = _B * _S
_NC = 2
_NS = 16
_NW = _NC * _NS
_BPW = _BTOT // _NW
_CHUNK = 8
_NBUF = 8
_NCHUNK = _BPW // _CHUNK
_GROUPS = _NCHUNK // _NBUF


def _emb_gather(pos_hbm, table_hbm, out_hbm, idx_v, rows_v, *sems):
    cid = lax.axis_index("c")
    sid = lax.axis_index("s")
    wid = sid * _NC + cid
    base = wid * _BPW
    pltpu.sync_copy(pos_hbm.at[pl.ds(base, _BPW)], idx_v)
    gsems = sems[:_NBUF]
    osems = sems[_NBUF:]

    def _aligned(off):
        return off if isinstance(off, int) else pl.multiple_of(off, _CHUNK)

    def gather_desc(g, b):
        return pltpu.make_async_copy(
            table_hbm.at[idx_v.at[pl.ds(_aligned(g * _CHUNK), _CHUNK)]],
            rows_v.at[b],
            gsems[b],
        )

    def store_desc(g, b):
        return pltpu.make_async_copy(
            rows_v.at[b],
            out_hbm.at[pl.ds(_aligned(base + g * _CHUNK), _CHUNK)],
            osems[b],
        )

    for b in range(_NBUF):
        gather_desc(b, b).start()

    def body(j, carry):
        for b in range(_NBUF):
            g = j * _NBUF + b
            gather_desc(g, b).wait()
            store_desc(g, b).start()
        for b in range(_NBUF):
            g = j * _NBUF + b
            store_desc(g, b).wait()
            gather_desc(g + _NBUF, b).start()
        return carry

    lax.fori_loop(0, _GROUPS - 1, body, 0)

    j = _GROUPS - 1
    for b in range(_NBUF):
        g = j * _NBUF + b
        gather_desc(g, b).wait()
        store_desc(g, b).start()
    for b in range(_NBUF):
        g = j * _NBUF + b
        store_desc(g, b).wait()


@jax.jit
def kernel(positions, table):
    pos_flat = positions.reshape(_BTOT).astype(jnp.int32)
    mesh = plsc.VectorSubcoreMesh(core_axis_name="c", subcore_axis_name="s")
    out = pl.kernel(
        _emb_gather,
        mesh=mesh,
        out_type=jax.ShapeDtypeStruct((_BTOT, _D), jnp.float32),
        scratch_types=[
            pltpu.VMEM((_BPW,), jnp.int32),
            pltpu.VMEM((_NBUF, _CHUNK, _D), jnp.float32),
        ] + [pltpu.SemaphoreType.DMA] * (2 * _NBUF),
    )(pos_flat, table)
    return out.reshape(_B, _S, _D)
